# Initial kernel scaffold; baseline (speedup 1.0000x reference)
import jax
import jax.numpy as jnp
from jax.experimental import pallas as pl


def kernel(x, edge_index, W_self1, W_neigh1, b1, W_self2, W_neigh2, b2):
    raise NotImplementedError("write your pallas kernel here")


# speedup vs baseline: 4.5997x; 4.5997x over previous
import jax
import jax.numpy as jnp
from jax import lax
from jax.experimental import pallas as pl
from jax.experimental.pallas import tpu as pltpu
from jax.experimental.pallas import tpu_sc as plsc

NC, NS, LANES = 2, 16, 16
NW = NC * NS
CHUNK = 80

_SC_PARAMS = pltpu.CompilerParams(use_tc_tiling_on_sc=False)


def _npt_npad(n):
    npt = -(-n // NS)
    npt = -(-npt // 32) * 32
    return npt, npt * NS


def _zero_fill(zbuf, zr, d):
    @pl.loop(0, zr)
    def _(i):
        for j in range(d // LANES):
            zbuf[i, pl.ds(j * LANES, LANES)] = jnp.zeros((LANES,), jnp.float32)


def _make_edge_agg_split(n, e, d):
    d2 = d // 2
    assert e % (NS * CHUNK) == 0
    iters = e // (NS * CHUNK)
    npt, npad = _npt_npad(n)
    zr = npt
    while zr * d2 * 4 > 128 * 1024:
        zr //= 2
    assert npt % zr == 0 and zr % 8 == 0

    mesh = plsc.VectorSubcoreMesh(core_axis_name="c", subcore_axis_name="s")
    out_type = [jax.ShapeDtypeStruct((NC, npad, d2), jnp.float32),
                jax.ShapeDtypeStruct((npad, LANES), jnp.float32)]
    scratch = [
        pltpu.VMEM((CHUNK,), jnp.int32),
        pltpu.VMEM((CHUNK,), jnp.int32),
        pltpu.VMEM((CHUNK,), jnp.int32),
        pltpu.VMEM((CHUNK, d2), jnp.float32),
        pltpu.VMEM((zr, d2), jnp.float32),
        pltpu.VMEM((CHUNK, LANES), jnp.float32),
        pltpu.VMEM((npt, LANES), jnp.float32),
        pltpu.VMEM_SHARED((npad, d2), jnp.float32),
        pltpu.VMEM_SHARED((npad, LANES), jnp.float32),
        pltpu.SemaphoreType.DMA,
    ]

    def body(feat_hbm, src_hbm, dst_hbm, agg_out, deg_out,
             sidx_v, gidx_v, didx_v, rows_v, zbuf, ones_v, zdeg, agg_sh,
             deg_sh, sem):
        c = lax.axis_index("c")
        s = lax.axis_index("s")

        _zero_fill(zbuf, zr, d2)
        for k in range(npt // zr):
            pltpu.sync_copy(zbuf, agg_sh.at[pl.ds(s * npt + k * zr, zr)])

        @pl.loop(0, CHUNK)
        def _(i):
            ones_v[i, :] = jnp.ones((LANES,), jnp.float32)

        @pl.loop(0, npt)
        def _(i):
            zdeg[i, :] = jnp.zeros((LANES,), jnp.float32)

        @pl.when(c == 0)
        def _():
            pltpu.sync_copy(zdeg, deg_sh.at[pl.ds(s * npt, npt)])

        plsc.subcore_barrier()

        base0 = s * iters * CHUNK

        @pl.loop(0, iters)
        def _(j):
            base = pl.multiple_of(base0 + j * CHUNK, CHUNK)
            pltpu.sync_copy(src_hbm.at[pl.ds(base, CHUNK)], sidx_v)
            pltpu.sync_copy(dst_hbm.at[pl.ds(base, CHUNK)], didx_v)
            for k in range(CHUNK // LANES):
                v = sidx_v[pl.ds(k * LANES, LANES)]
                gidx_v[pl.ds(k * LANES, LANES)] = v + v + c
            pltpu.async_copy(feat_hbm.at[gidx_v], rows_v, sem).wait()
            pltpu.sync_copy(rows_v, agg_sh.at[didx_v], add=True)

            @pl.when(c == 0)
            def _():
                pltpu.sync_copy(ones_v, deg_sh.at[didx_v], add=True)

        plsc.subcore_barrier()

        out0 = s * npt
        pltpu.sync_copy(agg_sh.at[pl.ds(out0, npt)],
                        agg_out.at[c, pl.ds(out0, npt)])

        @pl.when(c == 0)
        def _():
            pltpu.sync_copy(deg_sh.at[pl.ds(out0, npt)],
                            deg_out.at[pl.ds(out0, npt)])

    return pl.kernel(body, out_type=out_type, mesh=mesh,
                     scratch_types=scratch, compiler_params=_SC_PARAMS)


def _make_edge_agg(n, e, d):
    assert e % (NW * CHUNK) == 0
    iters = e // (NW * CHUNK)
    npt, npad = _npt_npad(n)
    zr = npt
    while zr * d * 4 > 128 * 1024:
        zr //= 2
    assert npt % zr == 0 and zr % 8 == 0

    mesh = plsc.VectorSubcoreMesh(core_axis_name="c", subcore_axis_name="s")
    out_type = [jax.ShapeDtypeStruct((NC, npad, d), jnp.float32)]
    scratch = [
        pltpu.VMEM((CHUNK,), jnp.int32),
        pltpu.VMEM((CHUNK,), jnp.int32),
        pltpu.VMEM((CHUNK, d), jnp.float32),
        pltpu.VMEM((zr, d), jnp.float32),
        pltpu.VMEM_SHARED((npad, d), jnp.float32),
        pltpu.SemaphoreType.DMA,
    ]

    def body(feat_hbm, src_hbm, dst_hbm, agg_out,
             sidx_v, didx_v, rows_v, zbuf, agg_sh, sem):
        c = lax.axis_index("c")
        s = lax.axis_index("s")
        w = c * NS + s

        _zero_fill(zbuf, zr, d)
        for k in range(npt // zr):
            pltpu.sync_copy(zbuf, agg_sh.at[pl.ds(s * npt + k * zr, zr)])

        plsc.subcore_barrier()

        base0 = w * iters * CHUNK

        @pl.loop(0, iters)
        def _(j):
            base = pl.multiple_of(base0 + j * CHUNK, CHUNK)
            pltpu.sync_copy(src_hbm.at[pl.ds(base, CHUNK)], sidx_v)
            pltpu.sync_copy(dst_hbm.at[pl.ds(base, CHUNK)], didx_v)
            pltpu.async_copy(feat_hbm.at[sidx_v], rows_v, sem).wait()
            pltpu.sync_copy(rows_v, agg_sh.at[didx_v], add=True)

        plsc.subcore_barrier()

        out0 = s * npt
        pltpu.sync_copy(agg_sh.at[pl.ds(out0, npt)],
                        agg_out.at[c, pl.ds(out0, npt)])

    return pl.kernel(body, out_type=out_type, mesh=mesh,
                     scratch_types=scratch, compiler_params=_SC_PARAMS)


def _dot_t(a, w):
    return lax.dot_general(a, w, (((1,), (1,)), ((), ())),
                           preferred_element_type=jnp.float32)


def _dense1_body(x_ref, alo_ref, ahi_ref, deg_ref, ws1_ref, wn1_ref,
                 b1_ref, ws2_ref, wn2_ref, h1_ref, h1r_ref, z_ref, s2_ref):
    inv = 1.0 / jnp.maximum(deg_ref[:, 0:1], 1.0)
    mean = jnp.concatenate([alo_ref[...], ahi_ref[...]], axis=1) * inv
    h1 = _dot_t(x_ref[...], ws1_ref[...]) + _dot_t(mean, wn1_ref[...]) + b1_ref[...]
    h1r = jnp.maximum(h1, 0.0)
    h1_ref[...] = h1
    h1r_ref[...] = h1r
    z_ref[...] = _dot_t(h1r, wn2_ref[...])
    s2_ref[...] = _dot_t(h1r, ws2_ref[...])


def _dense2_body(s2_ref, a0_ref, a1_ref, deg_ref, b2_ref, h2_ref):
    inv = 1.0 / jnp.maximum(deg_ref[:, 0:1], 1.0)
    h2_ref[...] = s2_ref[...] + (a0_ref[...] + a1_ref[...]) * inv + b2_ref[...]


def kernel(x, edge_index, W_self1, W_neigh1, b1, W_self2, W_neigh2, b2):
    n, d = x.shape
    h = W_self1.shape[0]
    cdim = W_self2.shape[0]
    e = edge_index.shape[1]

    src = edge_index[0]
    dst = edge_index[1]
    x2 = x.reshape(2 * n, d // 2)

    agg_fn = _make_edge_agg_split(n, e, d)
    aggp, deg = agg_fn(x2, src, dst)

    bn = 1000
    grid = (n // bn,)
    row_spec = lambda w: pl.BlockSpec((bn, w), lambda i: (i, 0))
    full_spec = lambda a, b: pl.BlockSpec((a, b), lambda i: (0, 0))

    h1, h1r, z, s2 = pl.pallas_call(
        _dense1_body,
        grid=grid,
        in_specs=[row_spec(d), row_spec(d // 2), row_spec(d // 2),
                  row_spec(LANES),
                  full_spec(h, d), full_spec(h, d), full_spec(1, h),
                  full_spec(cdim, h), full_spec(cdim, h)],
        out_specs=[row_spec(h), row_spec(h), row_spec(cdim), row_spec(cdim)],
        out_shape=[jax.ShapeDtypeStruct((n, h), jnp.float32),
                   jax.ShapeDtypeStruct((n, h), jnp.float32),
                   jax.ShapeDtypeStruct((n, cdim), jnp.float32),
                   jax.ShapeDtypeStruct((n, cdim), jnp.float32)],
    )(x, aggp[0], aggp[1], deg,
      W_self1, W_neigh1, b1.reshape(1, h), W_self2, W_neigh2)

    agg2_fn = _make_edge_agg(n, e, cdim)
    (agg2p,) = agg2_fn(z, src, dst)

    (h2,) = pl.pallas_call(
        _dense2_body,
        grid=grid,
        in_specs=[row_spec(cdim), row_spec(cdim), row_spec(cdim),
                  row_spec(LANES), full_spec(1, cdim)],
        out_specs=[row_spec(cdim)],
        out_shape=[jax.ShapeDtypeStruct((n, cdim), jnp.float32)],
    )(s2, agg2p[0], agg2p[1], deg, b2.reshape(1, cdim))

    return (h2, h1, h1r)


# speedup vs baseline: 11.1244x; 2.4185x over previous
import jax
import jax.numpy as jnp
from jax import lax
from jax.experimental import pallas as pl
from jax.experimental.pallas import tpu as pltpu
from jax.experimental.pallas import tpu_sc as plsc

NC, NS, LANES = 2, 16, 16
NW = NC * NS
CHUNK = 80

_SC_PARAMS = pltpu.CompilerParams(use_tc_tiling_on_sc=False)


def _npt_npad(n):
    npt = -(-n // NS)
    npt = -(-npt // 32) * 32
    return npt, npt * NS


def _zero_fill(zbuf, zr, d):
    @pl.loop(0, zr)
    def _(i):
        for j in range(d // LANES):
            zbuf[i, pl.ds(j * LANES, LANES)] = jnp.zeros((LANES,), jnp.float32)


def _zr_for(npt, d):
    zr = npt
    while zr * d * 4 > 32 * 1024:
        zr //= 2
    assert npt % zr == 0 and zr % 8 == 0
    return zr


def _make_edge_agg_split(n, e, d):
    d2 = d // 2
    assert e % (NS * CHUNK) == 0
    iters = e // (NS * CHUNK)
    npt, npad = _npt_npad(n)
    zr = _zr_for(npt, d2)
    zrd = _zr_for(npt, LANES)

    mesh = plsc.VectorSubcoreMesh(core_axis_name="c", subcore_axis_name="s")
    out_type = [jax.ShapeDtypeStruct((NC, npad, d2), jnp.float32),
                jax.ShapeDtypeStruct((NC, npad, LANES), jnp.float32)]
    idx = lambda: pltpu.VMEM((CHUNK,), jnp.int32)
    scratch = [
        idx(), idx(),
        idx(), idx(),
        idx(), idx(),
        idx(), idx(),
        pltpu.VMEM((CHUNK, d2), jnp.float32),
        pltpu.VMEM((CHUNK, d2), jnp.float32),
        pltpu.VMEM((zr, d2), jnp.float32),
        pltpu.VMEM((CHUNK, LANES), jnp.float32),
        pltpu.VMEM((zrd, LANES), jnp.float32),
        pltpu.VMEM_SHARED((npad, d2), jnp.float32),
        pltpu.VMEM_SHARED((npad, LANES), jnp.float32),
        pltpu.SemaphoreType.DMA, pltpu.SemaphoreType.DMA,
        pltpu.SemaphoreType.DMA, pltpu.SemaphoreType.DMA,
    ]

    def body(feat_hbm, src_hbm, dst_hbm, agg_out, deg_out,
             s0, s1, di0, di1, g0, g1, ds0, ds1, r0, r1, zbuf,
             ones_v, zdeg, agg_sh, deg_sh, iA, iB, gA, gB):
        c = lax.axis_index("c")
        s = lax.axis_index("s")
        sidx = (s0, s1)
        didx = (di0, di1)
        gidx = (g0, g1)
        didxS = (ds0, ds1)
        rows = (r0, r1)
        semI = (iA, iB)
        semG = (gA, gB)
        base0 = s * iters * CHUNK

        def issue_idx(m, p):
            off = pl.multiple_of(base0 + m * CHUNK, CHUNK)
            pltpu.async_copy(src_hbm.at[pl.ds(off, CHUNK)], sidx[p], semI[p])
            pltpu.async_copy(dst_hbm.at[pl.ds(off, CHUNK)], didx[p], semI[p])

        def wait_idx(p):
            pltpu.make_async_copy(src_hbm.at[pl.ds(0, CHUNK)], sidx[p], semI[p]).wait()
            pltpu.make_async_copy(dst_hbm.at[pl.ds(0, CHUNK)], didx[p], semI[p]).wait()

        def stage_and_gather(p):
            for k in range(CHUNK // LANES):
                v = sidx[p][pl.ds(k * LANES, LANES)]
                gidx[p][pl.ds(k * LANES, LANES)] = v + v + c
                didxS[p][pl.ds(k * LANES, LANES)] = didx[p][pl.ds(k * LANES, LANES)]
            pltpu.async_copy(feat_hbm.at[gidx[p]], rows[p], semG[p])

        def drain_scatter(p):
            pltpu.make_async_copy(feat_hbm.at[gidx[p]], rows[p], semG[p]).wait()
            pltpu.sync_copy(rows[p], agg_sh.at[didxS[p]], add=True)

            @pl.when(c == p)
            def _():
                pltpu.sync_copy(ones_v, deg_sh.at[didxS[p]], add=True)

        issue_idx(0, 0)
        _zero_fill(zbuf, zr, d2)
        for k in range(npt // zr):
            pltpu.sync_copy(zbuf, agg_sh.at[pl.ds(s * npt + k * zr, zr)])

        @pl.loop(0, CHUNK)
        def _(i):
            ones_v[i, :] = jnp.ones((LANES,), jnp.float32)

        _zero_fill(zdeg, zrd, LANES)
        for k in range(npt // zrd):
            pltpu.sync_copy(zdeg, deg_sh.at[pl.ds(s * npt + k * zrd, zrd)])

        plsc.subcore_barrier()

        wait_idx(0)
        stage_and_gather(0)
        issue_idx(1, 1)

        def position(m, p):
            @pl.when(m + 1 < iters)
            def _():
                wait_idx(1 - p)
                stage_and_gather(1 - p)

            @pl.when(m + 2 < iters)
            def _():
                issue_idx(m + 2, p)

            drain_scatter(p)

        @pl.loop(0, iters, step=2)
        def _(j):
            position(j, 0)

            @pl.when(j + 1 < iters)
            def _():
                position(j + 1, 1)

        plsc.subcore_barrier()

        out0 = s * npt
        pltpu.sync_copy(agg_sh.at[pl.ds(out0, npt)],
                        agg_out.at[c, pl.ds(out0, npt)])
        pltpu.sync_copy(deg_sh.at[pl.ds(out0, npt)],
                        deg_out.at[c, pl.ds(out0, npt)])

    return pl.kernel(body, out_type=out_type, mesh=mesh,
                     scratch_types=scratch, compiler_params=_SC_PARAMS)


def _make_edge_agg(n, e, d):
    assert e % (NW * CHUNK) == 0
    iters = e // (NW * CHUNK)
    npt, npad = _npt_npad(n)
    zr = _zr_for(npt, d)

    mesh = plsc.VectorSubcoreMesh(core_axis_name="c", subcore_axis_name="s")
    out_type = [jax.ShapeDtypeStruct((NC, npad, d), jnp.float32)]
    idx = lambda: pltpu.VMEM((CHUNK,), jnp.int32)
    scratch = [
        idx(), idx(),
        idx(), idx(),
        idx(), idx(),
        idx(), idx(),
        pltpu.VMEM((CHUNK, d), jnp.float32),
        pltpu.VMEM((CHUNK, d), jnp.float32),
        pltpu.VMEM((zr, d), jnp.float32),
        pltpu.VMEM_SHARED((npad, d), jnp.float32),
        pltpu.SemaphoreType.DMA, pltpu.SemaphoreType.DMA,
        pltpu.SemaphoreType.DMA, pltpu.SemaphoreType.DMA,
    ]

    def body(feat_hbm, src_hbm, dst_hbm, agg_out,
             s0, s1, di0, di1, g0, g1, ds0, ds1, r0, r1, zbuf,
             agg_sh, iA, iB, gA, gB):
        c = lax.axis_index("c")
        s = lax.axis_index("s")
        w = c * NS + s
        sidx = (s0, s1)
        didx = (di0, di1)
        gidx = (g0, g1)
        didxS = (ds0, ds1)
        rows = (r0, r1)
        semI = (iA, iB)
        semG = (gA, gB)
        base0 = w * iters * CHUNK

        def issue_idx(m, p):
            off = pl.multiple_of(base0 + m * CHUNK, CHUNK)
            pltpu.async_copy(src_hbm.at[pl.ds(off, CHUNK)], sidx[p], semI[p])
            pltpu.async_copy(dst_hbm.at[pl.ds(off, CHUNK)], didx[p], semI[p])

        def wait_idx(p):
            pltpu.make_async_copy(src_hbm.at[pl.ds(0, CHUNK)], sidx[p], semI[p]).wait()
            pltpu.make_async_copy(dst_hbm.at[pl.ds(0, CHUNK)], didx[p], semI[p]).wait()

        def stage_and_gather(p):
            for k in range(CHUNK // LANES):
                gidx[p][pl.ds(k * LANES, LANES)] = sidx[p][pl.ds(k * LANES, LANES)]
                didxS[p][pl.ds(k * LANES, LANES)] = didx[p][pl.ds(k * LANES, LANES)]
            pltpu.async_copy(feat_hbm.at[gidx[p]], rows[p], semG[p])

        def drain_scatter(p):
            pltpu.make_async_copy(feat_hbm.at[gidx[p]], rows[p], semG[p]).wait()
            pltpu.sync_copy(rows[p], agg_sh.at[didxS[p]], add=True)

        issue_idx(0, 0)
        _zero_fill(zbuf, zr, d)
        for k in range(npt // zr):
            pltpu.sync_copy(zbuf, agg_sh.at[pl.ds(s * npt + k * zr, zr)])

        plsc.subcore_barrier()

        wait_idx(0)
        stage_and_gather(0)
        issue_idx(1, 1)

        def position(m, p):
            @pl.when(m + 1 < iters)
            def _():
                wait_idx(1 - p)
                stage_and_gather(1 - p)

            @pl.when(m + 2 < iters)
            def _():
                issue_idx(m + 2, p)

            drain_scatter(p)

        @pl.loop(0, iters, step=2)
        def _(j):
            position(j, 0)

            @pl.when(j + 1 < iters)
            def _():
                position(j + 1, 1)

        plsc.subcore_barrier()

        out0 = s * npt
        pltpu.sync_copy(agg_sh.at[pl.ds(out0, npt)],
                        agg_out.at[c, pl.ds(out0, npt)])

    return pl.kernel(body, out_type=out_type, mesh=mesh,
                     scratch_types=scratch, compiler_params=_SC_PARAMS)


def _dot_t(a, w):
    return lax.dot_general(a, w, (((1,), (1,)), ((), ())),
                           preferred_element_type=jnp.float32)


def _dense1_body(x_ref, alo_ref, ahi_ref, dg0_ref, dg1_ref, ws1_ref, wn1_ref,
                 b1_ref, ws2_ref, wn2_ref, h1_ref, h1r_ref, z_ref, s2_ref):
    inv = 1.0 / jnp.maximum(dg0_ref[:, 0:1] + dg1_ref[:, 0:1], 1.0)
    mean = jnp.concatenate([alo_ref[...], ahi_ref[...]], axis=1) * inv
    h1 = _dot_t(x_ref[...], ws1_ref[...]) + _dot_t(mean, wn1_ref[...]) + b1_ref[...]
    h1r = jnp.maximum(h1, 0.0)
    h1_ref[...] = h1
    h1r_ref[...] = h1r
    z_ref[...] = _dot_t(h1r, wn2_ref[...])
    s2_ref[...] = _dot_t(h1r, ws2_ref[...])


def _dense2_body(s2_ref, a0_ref, a1_ref, dg0_ref, dg1_ref, b2_ref, h2_ref):
    inv = 1.0 / jnp.maximum(dg0_ref[:, 0:1] + dg1_ref[:, 0:1], 1.0)
    h2_ref[...] = s2_ref[...] + (a0_ref[...] + a1_ref[...]) * inv + b2_ref[...]


def kernel(x, edge_index, W_self1, W_neigh1, b1, W_self2, W_neigh2, b2):
    n, d = x.shape
    h = W_self1.shape[0]
    cdim = W_self2.shape[0]
    e = edge_index.shape[1]

    src = edge_index[0]
    dst = edge_index[1]
    x2 = x.reshape(2 * n, d // 2)

    agg_fn = _make_edge_agg_split(n, e, d)
    aggp, degp = agg_fn(x2, src, dst)

    bn = 1000
    grid = (n // bn,)
    row_spec = lambda w: pl.BlockSpec((bn, w), lambda i: (i, 0))
    full_spec = lambda a, b: pl.BlockSpec((a, b), lambda i: (0, 0))

    h1, h1r, z, s2 = pl.pallas_call(
        _dense1_body,
        grid=grid,
        in_specs=[row_spec(d), row_spec(d // 2), row_spec(d // 2),
                  row_spec(LANES), row_spec(LANES),
                  full_spec(h, d), full_spec(h, d), full_spec(1, h),
                  full_spec(cdim, h), full_spec(cdim, h)],
        out_specs=[row_spec(h), row_spec(h), row_spec(cdim), row_spec(cdim)],
        out_shape=[jax.ShapeDtypeStruct((n, h), jnp.float32),
                   jax.ShapeDtypeStruct((n, h), jnp.float32),
                   jax.ShapeDtypeStruct((n, cdim), jnp.float32),
                   jax.ShapeDtypeStruct((n, cdim), jnp.float32)],
    )(x, aggp[0], aggp[1], degp[0], degp[1],
      W_self1, W_neigh1, b1.reshape(1, h), W_self2, W_neigh2)

    agg2_fn = _make_edge_agg(n, e, cdim)
    (agg2p,) = agg2_fn(z, src, dst)

    (h2,) = pl.pallas_call(
        _dense2_body,
        grid=grid,
        in_specs=[row_spec(cdim), row_spec(cdim), row_spec(cdim),
                  row_spec(LANES), row_spec(LANES), full_spec(1, cdim)],
        out_specs=[row_spec(cdim)],
        out_shape=[jax.ShapeDtypeStruct((n, cdim), jnp.float32)],
    )(s2, agg2p[0], agg2p[1], degp[0], degp[1], b2.reshape(1, cdim))

    return (h2, h1, h1r)


# speedup vs baseline: 11.3616x; 1.0213x over previous
import jax
import jax.numpy as jnp
from jax import lax
from jax.experimental import pallas as pl
from jax.experimental.pallas import tpu as pltpu
from jax.experimental.pallas import tpu_sc as plsc

NC, NS, LANES = 2, 16, 16
NW = NC * NS
CHUNK = 80

_SC_PARAMS = pltpu.CompilerParams(use_tc_tiling_on_sc=False)


def _npt_npad(n):
    npt = -(-n // NS)
    npt = -(-npt // 32) * 32
    return npt, npt * NS


def _zero_fill(zbuf, zr, d):
    @pl.loop(0, zr)
    def _(i):
        for j in range(d // LANES):
            zbuf[i, pl.ds(j * LANES, LANES)] = jnp.zeros((LANES,), jnp.float32)


def _zr_for(npt, d):
    zr = npt
    while zr * d * 4 > 32 * 1024:
        zr //= 2
    assert npt % zr == 0 and zr % 8 == 0
    return zr


def _make_edge_agg_split(n, e, d):
    d2 = d // 2
    assert e % (NS * CHUNK) == 0
    iters = e // (NS * CHUNK)
    npt, npad = _npt_npad(n)
    zr = _zr_for(npt, d2)
    zrd = _zr_for(npt, LANES)

    mesh = plsc.VectorSubcoreMesh(core_axis_name="c", subcore_axis_name="s")
    out_type = [jax.ShapeDtypeStruct((NC, npad, d2), jnp.float32),
                jax.ShapeDtypeStruct((NC, npad, LANES), jnp.float32)]
    idx = lambda: pltpu.VMEM((CHUNK,), jnp.int32)
    scratch = [
        idx(), idx(),
        idx(), idx(),
        idx(), idx(),
        idx(), idx(),
        pltpu.VMEM((CHUNK, d2), jnp.float32),
        pltpu.VMEM((CHUNK, d2), jnp.float32),
        pltpu.VMEM((zr, d2), jnp.float32),
        pltpu.VMEM((CHUNK, LANES), jnp.float32),
        pltpu.VMEM((zrd, LANES), jnp.float32),
        pltpu.VMEM_SHARED((npad, d2), jnp.float32),
        pltpu.VMEM_SHARED((npad, LANES), jnp.float32),
        pltpu.SemaphoreType.DMA, pltpu.SemaphoreType.DMA,
        pltpu.SemaphoreType.DMA, pltpu.SemaphoreType.DMA,
        pltpu.SemaphoreType.DMA, pltpu.SemaphoreType.DMA,
    ]

    def body(feat_hbm, src_hbm, dst_hbm, agg_out, deg_out,
             s0, s1, di0, di1, g0, g1, ds0, ds1, r0, r1, zbuf,
             ones_v, zdeg, agg_sh, deg_sh, iA, iB, gA, gB, sA, sB):
        c = lax.axis_index("c")
        s = lax.axis_index("s")
        sidx = (s0, s1)
        didx = (di0, di1)
        gidx = (g0, g1)
        didxS = (ds0, ds1)
        rows = (r0, r1)
        semI = (iA, iB)
        semG = (gA, gB)
        semS = (sA, sB)
        base0 = s * iters * CHUNK

        def issue_idx(m, p):
            off = pl.multiple_of(base0 + m * CHUNK, CHUNK)
            pltpu.async_copy(src_hbm.at[pl.ds(off, CHUNK)], sidx[p], semI[p])
            pltpu.async_copy(dst_hbm.at[pl.ds(off, CHUNK)], didx[p], semI[p])

        def wait_idx(p):
            pltpu.make_async_copy(src_hbm.at[pl.ds(0, CHUNK)], sidx[p], semI[p]).wait()
            pltpu.make_async_copy(dst_hbm.at[pl.ds(0, CHUNK)], didx[p], semI[p]).wait()

        def stage_and_gather(p):
            for k in range(CHUNK // LANES):
                v = sidx[p][pl.ds(k * LANES, LANES)]
                gidx[p][pl.ds(k * LANES, LANES)] = v + v + c
                didxS[p][pl.ds(k * LANES, LANES)] = didx[p][pl.ds(k * LANES, LANES)]
            pltpu.async_copy(feat_hbm.at[gidx[p]], rows[p], semG[p])

        def issue_scatter(p):
            pltpu.make_async_copy(feat_hbm.at[gidx[p]], rows[p], semG[p]).wait()
            pltpu.async_copy(rows[p], agg_sh.at[didxS[p]], semS[p], add=True)

            @pl.when(c == p)
            def _():
                pltpu.async_copy(ones_v, deg_sh.at[didxS[p]], semS[p], add=True)

        def wait_scatter(p):
            pltpu.make_async_copy(rows[p], agg_sh.at[didxS[p]], semS[p]).wait()

            @pl.when(c == p)
            def _():
                pltpu.make_async_copy(ones_v, deg_sh.at[didxS[p]], semS[p]).wait()

        issue_idx(0, 0)
        _zero_fill(zbuf, zr, d2)
        for k in range(npt // zr):
            pltpu.sync_copy(zbuf, agg_sh.at[pl.ds(s * npt + k * zr, zr)])

        @pl.loop(0, CHUNK)
        def _(i):
            ones_v[i, :] = jnp.ones((LANES,), jnp.float32)

        _zero_fill(zdeg, zrd, LANES)
        for k in range(npt // zrd):
            pltpu.sync_copy(zdeg, deg_sh.at[pl.ds(s * npt + k * zrd, zrd)])

        plsc.subcore_barrier()

        wait_idx(0)
        stage_and_gather(0)
        issue_idx(1, 1)

        def position(m, p):
            @pl.when(m + 1 < iters)
            def _():
                wait_idx(1 - p)

                @pl.when(m >= 1)
                def _():
                    wait_scatter(1 - p)

                stage_and_gather(1 - p)

            @pl.when(m + 2 < iters)
            def _():
                issue_idx(m + 2, p)

            issue_scatter(p)

        @pl.loop(0, iters, step=2)
        def _(j):
            position(j, 0)

            @pl.when(j + 1 < iters)
            def _():
                position(j + 1, 1)

        wait_scatter(0)
        wait_scatter(1)
        plsc.subcore_barrier()

        out0 = s * npt
        pltpu.sync_copy(agg_sh.at[pl.ds(out0, npt)],
                        agg_out.at[c, pl.ds(out0, npt)])
        pltpu.sync_copy(deg_sh.at[pl.ds(out0, npt)],
                        deg_out.at[c, pl.ds(out0, npt)])

    return pl.kernel(body, out_type=out_type, mesh=mesh,
                     scratch_types=scratch, compiler_params=_SC_PARAMS)


def _make_edge_agg(n, e, d):
    assert e % (NW * CHUNK) == 0
    iters = e // (NW * CHUNK)
    npt, npad = _npt_npad(n)
    zr = _zr_for(npt, d)

    mesh = plsc.VectorSubcoreMesh(core_axis_name="c", subcore_axis_name="s")
    out_type = [jax.ShapeDtypeStruct((NC, npad, d), jnp.float32)]
    idx = lambda: pltpu.VMEM((CHUNK,), jnp.int32)
    scratch = [
        idx(), idx(),
        idx(), idx(),
        idx(), idx(),
        idx(), idx(),
        pltpu.VMEM((CHUNK, d), jnp.float32),
        pltpu.VMEM((CHUNK, d), jnp.float32),
        pltpu.VMEM((zr, d), jnp.float32),
        pltpu.VMEM_SHARED((npad, d), jnp.float32),
        pltpu.SemaphoreType.DMA, pltpu.SemaphoreType.DMA,
        pltpu.SemaphoreType.DMA, pltpu.SemaphoreType.DMA,
        pltpu.SemaphoreType.DMA, pltpu.SemaphoreType.DMA,
    ]

    def body(feat_hbm, src_hbm, dst_hbm, agg_out,
             s0, s1, di0, di1, g0, g1, ds0, ds1, r0, r1, zbuf,
             agg_sh, iA, iB, gA, gB, sA, sB):
        c = lax.axis_index("c")
        s = lax.axis_index("s")
        w = c * NS + s
        sidx = (s0, s1)
        didx = (di0, di1)
        gidx = (g0, g1)
        didxS = (ds0, ds1)
        rows = (r0, r1)
        semI = (iA, iB)
        semG = (gA, gB)
        semS = (sA, sB)
        base0 = w * iters * CHUNK

        def issue_idx(m, p):
            off = pl.multiple_of(base0 + m * CHUNK, CHUNK)
            pltpu.async_copy(src_hbm.at[pl.ds(off, CHUNK)], sidx[p], semI[p])
            pltpu.async_copy(dst_hbm.at[pl.ds(off, CHUNK)], didx[p], semI[p])

        def wait_idx(p):
            pltpu.make_async_copy(src_hbm.at[pl.ds(0, CHUNK)], sidx[p], semI[p]).wait()
            pltpu.make_async_copy(dst_hbm.at[pl.ds(0, CHUNK)], didx[p], semI[p]).wait()

        def stage_and_gather(p):
            for k in range(CHUNK // LANES):
                gidx[p][pl.ds(k * LANES, LANES)] = sidx[p][pl.ds(k * LANES, LANES)]
                didxS[p][pl.ds(k * LANES, LANES)] = didx[p][pl.ds(k * LANES, LANES)]
            pltpu.async_copy(feat_hbm.at[gidx[p]], rows[p], semG[p])

        def issue_scatter(p):
            pltpu.make_async_copy(feat_hbm.at[gidx[p]], rows[p], semG[p]).wait()
            pltpu.async_copy(rows[p], agg_sh.at[didxS[p]], semS[p], add=True)

        def wait_scatter(p):
            pltpu.make_async_copy(rows[p], agg_sh.at[didxS[p]], semS[p]).wait()

        issue_idx(0, 0)
        _zero_fill(zbuf, zr, d)
        for k in range(npt // zr):
            pltpu.sync_copy(zbuf, agg_sh.at[pl.ds(s * npt + k * zr, zr)])

        plsc.subcore_barrier()

        wait_idx(0)
        stage_and_gather(0)
        issue_idx(1, 1)

        def position(m, p):
            @pl.when(m + 1 < iters)
            def _():
                wait_idx(1 - p)

                @pl.when(m >= 1)
                def _():
                    wait_scatter(1 - p)

                stage_and_gather(1 - p)

            @pl.when(m + 2 < iters)
            def _():
                issue_idx(m + 2, p)

            issue_scatter(p)

        @pl.loop(0, iters, step=2)
        def _(j):
            position(j, 0)

            @pl.when(j + 1 < iters)
            def _():
                position(j + 1, 1)

        wait_scatter(0)
        wait_scatter(1)
        plsc.subcore_barrier()

        out0 = s * npt
        pltpu.sync_copy(agg_sh.at[pl.ds(out0, npt)],
                        agg_out.at[c, pl.ds(out0, npt)])

    return pl.kernel(body, out_type=out_type, mesh=mesh,
                     scratch_types=scratch, compiler_params=_SC_PARAMS)


def _dot_t(a, w):
    return lax.dot_general(a, w, (((1,), (1,)), ((), ())),
                           preferred_element_type=jnp.float32)


def _dense1_body(x_ref, alo_ref, ahi_ref, dg0_ref, dg1_ref, ws1_ref, wn1_ref,
                 b1_ref, ws2_ref, wn2_ref, h1_ref, h1r_ref, z_ref, s2_ref):
    inv = 1.0 / jnp.maximum(dg0_ref[:, 0:1] + dg1_ref[:, 0:1], 1.0)
    mean = jnp.concatenate([alo_ref[...], ahi_ref[...]], axis=1) * inv
    h1 = _dot_t(x_ref[...], ws1_ref[...]) + _dot_t(mean, wn1_ref[...]) + b1_ref[...]
    h1r = jnp.maximum(h1, 0.0)
    h1_ref[...] = h1
    h1r_ref[...] = h1r
    z_ref[...] = _dot_t(h1r, wn2_ref[...])
    s2_ref[...] = _dot_t(h1r, ws2_ref[...])


def _dense2_body(s2_ref, a0_ref, a1_ref, dg0_ref, dg1_ref, b2_ref, h2_ref):
    inv = 1.0 / jnp.maximum(dg0_ref[:, 0:1] + dg1_ref[:, 0:1], 1.0)
    h2_ref[...] = s2_ref[...] + (a0_ref[...] + a1_ref[...]) * inv + b2_ref[...]


def kernel(x, edge_index, W_self1, W_neigh1, b1, W_self2, W_neigh2, b2):
    n, d = x.shape
    h = W_self1.shape[0]
    cdim = W_self2.shape[0]
    e = edge_index.shape[1]

    src = edge_index[0]
    dst = edge_index[1]
    x2 = x.reshape(2 * n, d // 2)

    agg_fn = _make_edge_agg_split(n, e, d)
    aggp, degp = agg_fn(x2, src, dst)

    bn = 1000
    grid = (n // bn,)
    row_spec = lambda w: pl.BlockSpec((bn, w), lambda i: (i, 0))
    full_spec = lambda a, b: pl.BlockSpec((a, b), lambda i: (0, 0))

    h1, h1r, z, s2 = pl.pallas_call(
        _dense1_body,
        grid=grid,
        in_specs=[row_spec(d), row_spec(d // 2), row_spec(d // 2),
                  row_spec(LANES), row_spec(LANES),
                  full_spec(h, d), full_spec(h, d), full_spec(1, h),
                  full_spec(cdim, h), full_spec(cdim, h)],
        out_specs=[row_spec(h), row_spec(h), row_spec(cdim), row_spec(cdim)],
        out_shape=[jax.ShapeDtypeStruct((n, h), jnp.float32),
                   jax.ShapeDtypeStruct((n, h), jnp.float32),
                   jax.ShapeDtypeStruct((n, cdim), jnp.float32),
                   jax.ShapeDtypeStruct((n, cdim), jnp.float32)],
    )(x, aggp[0], aggp[1], degp[0], degp[1],
      W_self1, W_neigh1, b1.reshape(1, h), W_self2, W_neigh2)

    agg2_fn = _make_edge_agg(n, e, cdim)
    (agg2p,) = agg2_fn(z, src, dst)

    (h2,) = pl.pallas_call(
        _dense2_body,
        grid=grid,
        in_specs=[row_spec(cdim), row_spec(cdim), row_spec(cdim),
                  row_spec(LANES), row_spec(LANES), full_spec(1, cdim)],
        out_specs=[row_spec(cdim)],
        out_shape=[jax.ShapeDtypeStruct((n, cdim), jnp.float32)],
    )(s2, agg2p[0], agg2p[1], degp[0], degp[1], b2.reshape(1, cdim))

    return (h2, h1, h1r)


# speedup vs baseline: 16.8000x; 1.4787x over previous
import jax
import jax.numpy as jnp
from jax import lax
from jax.experimental import pallas as pl
from jax.experimental.pallas import tpu as pltpu
from jax.experimental.pallas import tpu_sc as plsc

NC, NS, LANES = 2, 16, 16
NW = NC * NS
CHUNK = 80

_SC_PARAMS = pltpu.CompilerParams(use_tc_tiling_on_sc=False)


def _npt_npad(n):
    npt = -(-n // NS)
    npt = -(-npt // 32) * 32
    return npt, npt * NS


def _zero_fill(zbuf, zr, d):
    @pl.loop(0, zr)
    def _(i):
        for j in range(d // LANES):
            zbuf[i, pl.ds(j * LANES, LANES)] = jnp.zeros((LANES,), jnp.float32)


def _zr_for(npt, d):
    zr = npt
    while zr * d * 4 > 32 * 1024:
        zr //= 2
    assert npt % zr == 0 and zr % 8 == 0
    return zr


def _make_edge_agg_split(n, e, d, nsub):
    d2 = d // 2
    sup = nsub * CHUNK
    assert e % (NS * sup) == 0
    iters = e // (NS * sup)
    npt, npad = _npt_npad(n)
    zr = _zr_for(npt, d2)
    zrd = _zr_for(npt, LANES)

    mesh = plsc.VectorSubcoreMesh(core_axis_name="c", subcore_axis_name="s")
    out_type = [jax.ShapeDtypeStruct((NC, npad, d2), jnp.float32),
                jax.ShapeDtypeStruct((NC, npad, LANES), jnp.float32)]
    scratch = [
        pltpu.VMEM((sup,), jnp.int32), pltpu.VMEM((sup,), jnp.int32),
        pltpu.VMEM((sup,), jnp.int32), pltpu.VMEM((sup,), jnp.int32),
    ]
    for _ in range(2 * nsub):
        scratch.append(pltpu.VMEM((CHUNK,), jnp.int32))
    for _ in range(2 * nsub):
        scratch.append(pltpu.VMEM((CHUNK,), jnp.int32))
    for _ in range(2 * nsub):
        scratch.append(pltpu.VMEM((CHUNK, d2), jnp.float32))
    scratch += [
        pltpu.VMEM((zr, d2), jnp.float32),
        pltpu.VMEM((CHUNK, LANES), jnp.float32),
        pltpu.VMEM((zrd, LANES), jnp.float32),
        pltpu.VMEM_SHARED((npad, d2), jnp.float32),
        pltpu.VMEM_SHARED((npad, LANES), jnp.float32),
        pltpu.SemaphoreType.DMA, pltpu.SemaphoreType.DMA,
        pltpu.SemaphoreType.DMA, pltpu.SemaphoreType.DMA,
        pltpu.SemaphoreType.DMA, pltpu.SemaphoreType.DMA,
    ]

    def body(feat_hbm, src_hbm, dst_hbm, agg_out, deg_out, *refs):
        sidx = refs[0:2]
        didx = refs[2:4]
        gidx = (refs[4:4 + nsub], refs[4 + nsub:4 + 2 * nsub])
        o = 4 + 2 * nsub
        didxS = (refs[o:o + nsub], refs[o + nsub:o + 2 * nsub])
        o += 2 * nsub
        rows = (refs[o:o + nsub], refs[o + nsub:o + 2 * nsub])
        o += 2 * nsub
        zbuf, ones_v, zdeg, agg_sh, deg_sh = refs[o:o + 5]
        semI = refs[o + 5:o + 7]
        semG = refs[o + 7:o + 9]
        semS = refs[o + 9:o + 11]

        c = lax.axis_index("c")
        s = lax.axis_index("s")
        base0 = s * iters * sup

        def issue_idx(m, p):
            off = pl.multiple_of(base0 + m * sup, CHUNK)
            pltpu.async_copy(src_hbm.at[pl.ds(off, sup)], sidx[p], semI[p])
            pltpu.async_copy(dst_hbm.at[pl.ds(off, sup)], didx[p], semI[p])

        def wait_idx(p):
            pltpu.make_async_copy(src_hbm.at[pl.ds(0, sup)], sidx[p], semI[p]).wait()
            pltpu.make_async_copy(dst_hbm.at[pl.ds(0, sup)], didx[p], semI[p]).wait()

        def stage_and_gather(p):
            for u in range(nsub):
                for k in range(CHUNK // LANES):
                    off = u * CHUNK + k * LANES
                    v = sidx[p][pl.ds(off, LANES)]
                    gidx[p][u][pl.ds(k * LANES, LANES)] = v + v + c
                    didxS[p][u][pl.ds(k * LANES, LANES)] = didx[p][pl.ds(off, LANES)]
                pltpu.async_copy(feat_hbm.at[gidx[p][u]], rows[p][u], semG[p])

        def issue_scatter(p):
            for u in range(nsub):
                pltpu.make_async_copy(feat_hbm.at[gidx[p][u]], rows[p][u],
                                      semG[p]).wait()
                pltpu.async_copy(rows[p][u], agg_sh.at[didxS[p][u]], semS[p],
                                 add=True)

            @pl.when(c == p)
            def _():
                for u in range(nsub):
                    pltpu.async_copy(ones_v, deg_sh.at[didxS[p][u]], semS[p],
                                     add=True)

        def wait_scatter(p):
            for u in range(nsub):
                pltpu.make_async_copy(rows[p][u], agg_sh.at[didxS[p][u]],
                                      semS[p]).wait()

            @pl.when(c == p)
            def _():
                for u in range(nsub):
                    pltpu.make_async_copy(ones_v, deg_sh.at[didxS[p][u]],
                                          semS[p]).wait()

        issue_idx(0, 0)
        _zero_fill(zbuf, zr, d2)
        for k in range(npt // zr):
            pltpu.sync_copy(zbuf, agg_sh.at[pl.ds(s * npt + k * zr, zr)])

        @pl.loop(0, CHUNK)
        def _(i):
            ones_v[i, :] = jnp.ones((LANES,), jnp.float32)

        _zero_fill(zdeg, zrd, LANES)
        for k in range(npt // zrd):
            pltpu.sync_copy(zdeg, deg_sh.at[pl.ds(s * npt + k * zrd, zrd)])

        plsc.subcore_barrier()

        wait_idx(0)
        stage_and_gather(0)
        issue_idx(1, 1)

        def position(m, p):
            @pl.when(m + 1 < iters)
            def _():
                wait_idx(1 - p)

                @pl.when(m >= 1)
                def _():
                    wait_scatter(1 - p)

                stage_and_gather(1 - p)

            @pl.when(m + 2 < iters)
            def _():
                issue_idx(m + 2, p)

            issue_scatter(p)

        @pl.loop(0, iters, step=2)
        def _(j):
            position(j, 0)

            @pl.when(j + 1 < iters)
            def _():
                position(j + 1, 1)

        wait_scatter(0)
        wait_scatter(1)
        plsc.subcore_barrier()

        out0 = s * npt
        pltpu.sync_copy(agg_sh.at[pl.ds(out0, npt)],
                        agg_out.at[c, pl.ds(out0, npt)])
        pltpu.sync_copy(deg_sh.at[pl.ds(out0, npt)],
                        deg_out.at[c, pl.ds(out0, npt)])

    return pl.kernel(body, out_type=out_type, mesh=mesh,
                     scratch_types=scratch, compiler_params=_SC_PARAMS)


def _make_edge_agg(n, e, d, nsub):
    sup = nsub * CHUNK
    assert e % (NW * sup) == 0
    iters = e // (NW * sup)
    npt, npad = _npt_npad(n)
    zr = _zr_for(npt, d)

    mesh = plsc.VectorSubcoreMesh(core_axis_name="c", subcore_axis_name="s")
    out_type = [jax.ShapeDtypeStruct((NC, npad, d), jnp.float32)]
    scratch = [
        pltpu.VMEM((sup,), jnp.int32), pltpu.VMEM((sup,), jnp.int32),
        pltpu.VMEM((sup,), jnp.int32), pltpu.VMEM((sup,), jnp.int32),
    ]
    for _ in range(2 * nsub):
        scratch.append(pltpu.VMEM((CHUNK,), jnp.int32))
    for _ in range(2 * nsub):
        scratch.append(pltpu.VMEM((CHUNK,), jnp.int32))
    for _ in range(2 * nsub):
        scratch.append(pltpu.VMEM((CHUNK, d), jnp.float32))
    scratch += [
        pltpu.VMEM((zr, d), jnp.float32),
        pltpu.VMEM_SHARED((npad, d), jnp.float32),
        pltpu.SemaphoreType.DMA, pltpu.SemaphoreType.DMA,
        pltpu.SemaphoreType.DMA, pltpu.SemaphoreType.DMA,
        pltpu.SemaphoreType.DMA, pltpu.SemaphoreType.DMA,
    ]

    def body(feat_hbm, src_hbm, dst_hbm, agg_out, *refs):
        sidx = refs[0:2]
        didx = refs[2:4]
        gidx = (refs[4:4 + nsub], refs[4 + nsub:4 + 2 * nsub])
        o = 4 + 2 * nsub
        didxS = (refs[o:o + nsub], refs[o + nsub:o + 2 * nsub])
        o += 2 * nsub
        rows = (refs[o:o + nsub], refs[o + nsub:o + 2 * nsub])
        o += 2 * nsub
        zbuf, agg_sh = refs[o:o + 2]
        semI = refs[o + 2:o + 4]
        semG = refs[o + 4:o + 6]
        semS = refs[o + 6:o + 8]

        c = lax.axis_index("c")
        s = lax.axis_index("s")
        w = c * NS + s
        base0 = w * iters * sup

        def issue_idx(m, p):
            off = pl.multiple_of(base0 + m * sup, CHUNK)
            pltpu.async_copy(src_hbm.at[pl.ds(off, sup)], sidx[p], semI[p])
            pltpu.async_copy(dst_hbm.at[pl.ds(off, sup)], didx[p], semI[p])

        def wait_idx(p):
            pltpu.make_async_copy(src_hbm.at[pl.ds(0, sup)], sidx[p], semI[p]).wait()
            pltpu.make_async_copy(dst_hbm.at[pl.ds(0, sup)], didx[p], semI[p]).wait()

        def stage_and_gather(p):
            for u in range(nsub):
                for k in range(CHUNK // LANES):
                    off = u * CHUNK + k * LANES
                    gidx[p][u][pl.ds(k * LANES, LANES)] = sidx[p][pl.ds(off, LANES)]
                    didxS[p][u][pl.ds(k * LANES, LANES)] = didx[p][pl.ds(off, LANES)]
                pltpu.async_copy(feat_hbm.at[gidx[p][u]], rows[p][u], semG[p])

        def issue_scatter(p):
            for u in range(nsub):
                pltpu.make_async_copy(feat_hbm.at[gidx[p][u]], rows[p][u],
                                      semG[p]).wait()
                pltpu.async_copy(rows[p][u], agg_sh.at[didxS[p][u]], semS[p],
                                 add=True)

        def wait_scatter(p):
            for u in range(nsub):
                pltpu.make_async_copy(rows[p][u], agg_sh.at[didxS[p][u]],
                                      semS[p]).wait()

        issue_idx(0, 0)
        _zero_fill(zbuf, zr, d)
        for k in range(npt // zr):
            pltpu.sync_copy(zbuf, agg_sh.at[pl.ds(s * npt + k * zr, zr)])

        plsc.subcore_barrier()

        wait_idx(0)
        stage_and_gather(0)
        issue_idx(1, 1)

        def position(m, p):
            @pl.when(m + 1 < iters)
            def _():
                wait_idx(1 - p)

                @pl.when(m >= 1)
                def _():
                    wait_scatter(1 - p)

                stage_and_gather(1 - p)

            @pl.when(m + 2 < iters)
            def _():
                issue_idx(m + 2, p)

            issue_scatter(p)

        @pl.loop(0, iters, step=2)
        def _(j):
            position(j, 0)

            @pl.when(j + 1 < iters)
            def _():
                position(j + 1, 1)

        wait_scatter(0)
        wait_scatter(1)
        plsc.subcore_barrier()

        out0 = s * npt
        pltpu.sync_copy(agg_sh.at[pl.ds(out0, npt)],
                        agg_out.at[c, pl.ds(out0, npt)])

    return pl.kernel(body, out_type=out_type, mesh=mesh,
                     scratch_types=scratch, compiler_params=_SC_PARAMS)


def _dot_t(a, w):
    return lax.dot_general(a, w, (((1,), (1,)), ((), ())),
                           preferred_element_type=jnp.float32)


def _dense1_body(x_ref, agg_ref, deg_ref, ws1_ref, wn1_ref,
                 b1_ref, ws2_ref, wn2_ref, h1_ref, h1r_ref, z_ref, s2_ref):
    inv = 1.0 / jnp.maximum(deg_ref[0, :, 0:1] + deg_ref[1, :, 0:1], 1.0)
    mean = jnp.concatenate([agg_ref[0], agg_ref[1]], axis=1) * inv
    h1 = _dot_t(x_ref[...], ws1_ref[...]) + _dot_t(mean, wn1_ref[...]) + b1_ref[...]
    h1r = jnp.maximum(h1, 0.0)
    h1_ref[...] = h1
    h1r_ref[...] = h1r
    z_ref[...] = _dot_t(h1r, wn2_ref[...])
    s2_ref[...] = _dot_t(h1r, ws2_ref[...])


def _dense2_body(s2_ref, a2_ref, deg_ref, b2_ref, h2_ref):
    inv = 1.0 / jnp.maximum(deg_ref[0, :, 0:1] + deg_ref[1, :, 0:1], 1.0)
    h2_ref[...] = s2_ref[...] + (a2_ref[0] + a2_ref[1]) * inv + b2_ref[...]


def kernel(x, edge_index, W_self1, W_neigh1, b1, W_self2, W_neigh2, b2):
    n, d = x.shape
    h = W_self1.shape[0]
    cdim = W_self2.shape[0]
    e = edge_index.shape[1]

    src = edge_index[0]
    dst = edge_index[1]
    x2 = x.reshape(2 * n, d // 2)

    agg_fn = _make_edge_agg_split(n, e, d, nsub=2)
    aggp, degp = agg_fn(x2, src, dst)

    bn = 1000
    grid = (n // bn,)
    row_spec = lambda w: pl.BlockSpec((bn, w), lambda i: (i, 0))
    part_spec = lambda w: pl.BlockSpec((NC, bn, w), lambda i: (0, i, 0))
    full_spec = lambda a, b: pl.BlockSpec((a, b), lambda i: (0, 0))

    h1, h1r, z, s2 = pl.pallas_call(
        _dense1_body,
        grid=grid,
        in_specs=[row_spec(d), part_spec(d // 2), part_spec(LANES),
                  full_spec(h, d), full_spec(h, d), full_spec(1, h),
                  full_spec(cdim, h), full_spec(cdim, h)],
        out_specs=[row_spec(h), row_spec(h), row_spec(cdim), row_spec(cdim)],
        out_shape=[jax.ShapeDtypeStruct((n, h), jnp.float32),
                   jax.ShapeDtypeStruct((n, h), jnp.float32),
                   jax.ShapeDtypeStruct((n, cdim), jnp.float32),
                   jax.ShapeDtypeStruct((n, cdim), jnp.float32)],
    )(x, aggp, degp, W_self1, W_neigh1, b1.reshape(1, h), W_self2, W_neigh2)

    agg2_fn = _make_edge_agg(n, e, cdim, nsub=5)
    (agg2p,) = agg2_fn(z, src, dst)

    (h2,) = pl.pallas_call(
        _dense2_body,
        grid=grid,
        in_specs=[row_spec(cdim), part_spec(cdim), part_spec(LANES),
                  full_spec(1, cdim)],
        out_specs=[row_spec(cdim)],
        out_shape=[jax.ShapeDtypeStruct((n, cdim), jnp.float32)],
    )(s2, agg2p, degp, b2.reshape(1, cdim))

    return (h2, h1, h1r)


# speedup vs baseline: 17.6849x; 1.0527x over previous
import jax
import jax.numpy as jnp
from jax import lax
from jax.experimental import pallas as pl
from jax.experimental.pallas import tpu as pltpu
from jax.experimental.pallas import tpu_sc as plsc

NC, NS, LANES = 2, 16, 16
NW = NC * NS
CHUNK = 80

_SC_PARAMS = pltpu.CompilerParams(use_tc_tiling_on_sc=False)


def _npt_npad(n):
    npt = -(-n // NS)
    npt = -(-npt // 32) * 32
    return npt, npt * NS


def _zero_fill(zbuf, zr, d):
    @pl.loop(0, zr)
    def _(i):
        for j in range(d // LANES):
            zbuf[i, pl.ds(j * LANES, LANES)] = jnp.zeros((LANES,), jnp.float32)


def _zr_for(npt, d):
    zr = npt
    while zr * d * 4 > 32 * 1024:
        zr //= 2
    assert npt % zr == 0 and zr % 8 == 0
    return zr


def _make_edge_agg_split(n, e, d, nsub):
    d2 = d // 2
    sup = nsub * CHUNK
    assert e % (NS * sup) == 0
    iters = e // (NS * sup)
    npt, npad = _npt_npad(n)
    zr = _zr_for(npt, d2)
    zrd = _zr_for(npt, LANES)

    mesh = plsc.VectorSubcoreMesh(core_axis_name="c", subcore_axis_name="s")
    out_type = [jax.ShapeDtypeStruct((NC, npad, d2), jnp.float32),
                jax.ShapeDtypeStruct((NC, npad, LANES), jnp.float32)]
    scratch = [
        pltpu.VMEM((sup,), jnp.int32), pltpu.VMEM((sup,), jnp.int32),
        pltpu.VMEM((sup,), jnp.int32), pltpu.VMEM((sup,), jnp.int32),
    ]
    for _ in range(2 * nsub):
        scratch.append(pltpu.VMEM((CHUNK,), jnp.int32))
    for _ in range(2 * nsub):
        scratch.append(pltpu.VMEM((CHUNK,), jnp.int32))
    for _ in range(2 * nsub):
        scratch.append(pltpu.VMEM((CHUNK, d2), jnp.float32))
    scratch += [
        pltpu.VMEM((zr, d2), jnp.float32),
        pltpu.VMEM((CHUNK, LANES), jnp.float32),
        pltpu.VMEM((zrd, LANES), jnp.float32),
        pltpu.VMEM_SHARED((npad, d2), jnp.float32),
        pltpu.VMEM_SHARED((npad, LANES), jnp.float32),
        pltpu.SemaphoreType.DMA, pltpu.SemaphoreType.DMA,
        pltpu.SemaphoreType.DMA, pltpu.SemaphoreType.DMA,
        pltpu.SemaphoreType.DMA, pltpu.SemaphoreType.DMA,
    ]

    def body(feat_hbm, src_hbm, dst_hbm, agg_out, deg_out, *refs):
        sidx = refs[0:2]
        didx = refs[2:4]
        gidx = (refs[4:4 + nsub], refs[4 + nsub:4 + 2 * nsub])
        o = 4 + 2 * nsub
        didxS = (refs[o:o + nsub], refs[o + nsub:o + 2 * nsub])
        o += 2 * nsub
        rows = (refs[o:o + nsub], refs[o + nsub:o + 2 * nsub])
        o += 2 * nsub
        zbuf, ones_v, zdeg, agg_sh, deg_sh = refs[o:o + 5]
        semI = refs[o + 5:o + 7]
        semG = refs[o + 7:o + 9]
        semS = refs[o + 9:o + 11]

        c = lax.axis_index("c")
        s = lax.axis_index("s")
        base0 = s * iters * sup

        def issue_idx(m, p):
            off = pl.multiple_of(base0 + m * sup, CHUNK)
            pltpu.async_copy(src_hbm.at[pl.ds(off, sup)], sidx[p], semI[p])
            pltpu.async_copy(dst_hbm.at[pl.ds(off, sup)], didx[p], semI[p])

        def wait_idx(p):
            pltpu.make_async_copy(src_hbm.at[pl.ds(0, sup)], sidx[p], semI[p]).wait()
            pltpu.make_async_copy(dst_hbm.at[pl.ds(0, sup)], didx[p], semI[p]).wait()

        def stage_and_gather(p):
            for u in range(nsub):
                for k in range(CHUNK // LANES):
                    off = u * CHUNK + k * LANES
                    v = sidx[p][pl.ds(off, LANES)]
                    gidx[p][u][pl.ds(k * LANES, LANES)] = v + v + c
                    didxS[p][u][pl.ds(k * LANES, LANES)] = didx[p][pl.ds(off, LANES)]
                pltpu.async_copy(feat_hbm.at[gidx[p][u]], rows[p][u], semG[p])

        def issue_scatter(p):
            for u in range(nsub):
                pltpu.make_async_copy(feat_hbm.at[gidx[p][u]], rows[p][u],
                                      semG[p]).wait()
                pltpu.async_copy(rows[p][u], agg_sh.at[didxS[p][u]], semS[p],
                                 add=True)

            @pl.when(c == p)
            def _():
                for u in range(nsub):
                    pltpu.async_copy(ones_v, deg_sh.at[didxS[p][u]], semS[p],
                                     add=True)

        def wait_scatter(p):
            for u in range(nsub):
                pltpu.make_async_copy(rows[p][u], agg_sh.at[didxS[p][u]],
                                      semS[p]).wait()

            @pl.when(c == p)
            def _():
                for u in range(nsub):
                    pltpu.make_async_copy(ones_v, deg_sh.at[didxS[p][u]],
                                          semS[p]).wait()

        issue_idx(0, 0)
        _zero_fill(zbuf, zr, d2)
        for k in range(npt // zr):
            pltpu.sync_copy(zbuf, agg_sh.at[pl.ds(s * npt + k * zr, zr)])

        @pl.loop(0, CHUNK)
        def _(i):
            ones_v[i, :] = jnp.ones((LANES,), jnp.float32)

        _zero_fill(zdeg, zrd, LANES)
        for k in range(npt // zrd):
            pltpu.sync_copy(zdeg, deg_sh.at[pl.ds(s * npt + k * zrd, zrd)])

        plsc.subcore_barrier()

        wait_idx(0)
        stage_and_gather(0)
        issue_idx(1, 1)

        def position(m, p):
            @pl.when(m + 1 < iters)
            def _():
                wait_idx(1 - p)

                @pl.when(m >= 1)
                def _():
                    wait_scatter(1 - p)

                stage_and_gather(1 - p)

            @pl.when(m + 2 < iters)
            def _():
                issue_idx(m + 2, p)

            issue_scatter(p)

        @pl.loop(0, iters, step=2)
        def _(j):
            position(j, 0)

            @pl.when(j + 1 < iters)
            def _():
                position(j + 1, 1)

        wait_scatter(0)
        wait_scatter(1)
        plsc.subcore_barrier()

        out0 = s * npt
        pltpu.sync_copy(agg_sh.at[pl.ds(out0, npt)],
                        agg_out.at[c, pl.ds(out0, npt)])
        pltpu.sync_copy(deg_sh.at[pl.ds(out0, npt)],
                        deg_out.at[c, pl.ds(out0, npt)])

    return pl.kernel(body, out_type=out_type, mesh=mesh,
                     scratch_types=scratch, compiler_params=_SC_PARAMS)


def _make_edge_agg(n, e, d, nsub):
    sup = nsub * CHUNK
    assert e % (NW * sup) == 0
    iters = e // (NW * sup)
    npt, npad = _npt_npad(n)
    zr = _zr_for(npt, d)

    mesh = plsc.VectorSubcoreMesh(core_axis_name="c", subcore_axis_name="s")
    out_type = [jax.ShapeDtypeStruct((NC, npad, d), jnp.float32)]
    scratch = [
        pltpu.VMEM((sup,), jnp.int32), pltpu.VMEM((sup,), jnp.int32),
        pltpu.VMEM((sup,), jnp.int32), pltpu.VMEM((sup,), jnp.int32),
    ]
    for _ in range(2 * nsub):
        scratch.append(pltpu.VMEM((CHUNK,), jnp.int32))
    for _ in range(2 * nsub):
        scratch.append(pltpu.VMEM((CHUNK,), jnp.int32))
    for _ in range(2 * nsub):
        scratch.append(pltpu.VMEM((CHUNK, d), jnp.float32))
    scratch += [
        pltpu.VMEM((zr, d), jnp.float32),
        pltpu.VMEM_SHARED((npad, d), jnp.float32),
        pltpu.SemaphoreType.DMA, pltpu.SemaphoreType.DMA,
        pltpu.SemaphoreType.DMA, pltpu.SemaphoreType.DMA,
        pltpu.SemaphoreType.DMA, pltpu.SemaphoreType.DMA,
    ]

    def body(feat_hbm, src_hbm, dst_hbm, agg_out, *refs):
        sidx = refs[0:2]
        didx = refs[2:4]
        gidx = (refs[4:4 + nsub], refs[4 + nsub:4 + 2 * nsub])
        o = 4 + 2 * nsub
        didxS = (refs[o:o + nsub], refs[o + nsub:o + 2 * nsub])
        o += 2 * nsub
        rows = (refs[o:o + nsub], refs[o + nsub:o + 2 * nsub])
        o += 2 * nsub
        zbuf, agg_sh = refs[o:o + 2]
        semI = refs[o + 2:o + 4]
        semG = refs[o + 4:o + 6]
        semS = refs[o + 6:o + 8]

        c = lax.axis_index("c")
        s = lax.axis_index("s")
        w = c * NS + s
        base0 = w * iters * sup

        def issue_idx(m, p):
            off = pl.multiple_of(base0 + m * sup, CHUNK)
            pltpu.async_copy(src_hbm.at[pl.ds(off, sup)], sidx[p], semI[p])
            pltpu.async_copy(dst_hbm.at[pl.ds(off, sup)], didx[p], semI[p])

        def wait_idx(p):
            pltpu.make_async_copy(src_hbm.at[pl.ds(0, sup)], sidx[p], semI[p]).wait()
            pltpu.make_async_copy(dst_hbm.at[pl.ds(0, sup)], didx[p], semI[p]).wait()

        def stage_and_gather(p):
            for u in range(nsub):
                for k in range(CHUNK // LANES):
                    off = u * CHUNK + k * LANES
                    gidx[p][u][pl.ds(k * LANES, LANES)] = sidx[p][pl.ds(off, LANES)]
                    didxS[p][u][pl.ds(k * LANES, LANES)] = didx[p][pl.ds(off, LANES)]
                pltpu.async_copy(feat_hbm.at[gidx[p][u]], rows[p][u], semG[p])

        def issue_scatter(p):
            for u in range(nsub):
                pltpu.make_async_copy(feat_hbm.at[gidx[p][u]], rows[p][u],
                                      semG[p]).wait()
                pltpu.async_copy(rows[p][u], agg_sh.at[didxS[p][u]], semS[p],
                                 add=True)

        def wait_scatter(p):
            for u in range(nsub):
                pltpu.make_async_copy(rows[p][u], agg_sh.at[didxS[p][u]],
                                      semS[p]).wait()

        issue_idx(0, 0)
        _zero_fill(zbuf, zr, d)
        for k in range(npt // zr):
            pltpu.sync_copy(zbuf, agg_sh.at[pl.ds(s * npt + k * zr, zr)])

        plsc.subcore_barrier()

        wait_idx(0)
        stage_and_gather(0)
        issue_idx(1, 1)

        def position(m, p):
            @pl.when(m + 1 < iters)
            def _():
                wait_idx(1 - p)

                @pl.when(m >= 1)
                def _():
                    wait_scatter(1 - p)

                stage_and_gather(1 - p)

            @pl.when(m + 2 < iters)
            def _():
                issue_idx(m + 2, p)

            issue_scatter(p)

        @pl.loop(0, iters, step=2)
        def _(j):
            position(j, 0)

            @pl.when(j + 1 < iters)
            def _():
                position(j + 1, 1)

        wait_scatter(0)
        wait_scatter(1)
        plsc.subcore_barrier()

        out0 = s * npt
        pltpu.sync_copy(agg_sh.at[pl.ds(out0, npt)],
                        agg_out.at[c, pl.ds(out0, npt)])

    return pl.kernel(body, out_type=out_type, mesh=mesh,
                     scratch_types=scratch, compiler_params=_SC_PARAMS)


def _dot_t(a, w):
    return lax.dot_general(a, w, (((1,), (1,)), ((), ())),
                           preferred_element_type=jnp.float32)


def _dense1_body(x_ref, agg_ref, deg_ref, ws1_ref, wn1_ref,
                 b1_ref, ws2_ref, wn2_ref, h1_ref, h1r_ref, z_ref, s2_ref):
    inv = 1.0 / jnp.maximum(deg_ref[0, :, 0:1] + deg_ref[1, :, 0:1], 1.0)
    mean = jnp.concatenate([agg_ref[0], agg_ref[1]], axis=1) * inv
    h1 = _dot_t(x_ref[...], ws1_ref[...]) + _dot_t(mean, wn1_ref[...]) + b1_ref[...]
    h1r = jnp.maximum(h1, 0.0)
    h1_ref[...] = h1
    h1r_ref[...] = h1r
    z_ref[...] = _dot_t(h1r, wn2_ref[...])
    s2_ref[...] = _dot_t(h1r, ws2_ref[...])


def _dense2_body(s2_ref, a2_ref, deg_ref, b2_ref, h2_ref):
    inv = 1.0 / jnp.maximum(deg_ref[0, :, 0:1] + deg_ref[1, :, 0:1], 1.0)
    h2_ref[...] = s2_ref[...] + (a2_ref[0] + a2_ref[1]) * inv + b2_ref[...]


def kernel(x, edge_index, W_self1, W_neigh1, b1, W_self2, W_neigh2, b2):
    n, d = x.shape
    h = W_self1.shape[0]
    cdim = W_self2.shape[0]
    e = edge_index.shape[1]

    src = edge_index[0]
    dst = edge_index[1]
    x2 = x.reshape(2 * n, d // 2)

    agg_fn = _make_edge_agg_split(n, e, d, nsub=5)
    aggp, degp = agg_fn(x2, src, dst)

    bn = 1000
    grid = (n // bn,)
    row_spec = lambda w: pl.BlockSpec((bn, w), lambda i: (i, 0))
    part_spec = lambda w: pl.BlockSpec((NC, bn, w), lambda i: (0, i, 0))
    full_spec = lambda a, b: pl.BlockSpec((a, b), lambda i: (0, 0))

    h1, h1r, z, s2 = pl.pallas_call(
        _dense1_body,
        grid=grid,
        in_specs=[row_spec(d), part_spec(d // 2), part_spec(LANES),
                  full_spec(h, d), full_spec(h, d), full_spec(1, h),
                  full_spec(cdim, h), full_spec(cdim, h)],
        out_specs=[row_spec(h), row_spec(h), row_spec(cdim), row_spec(cdim)],
        out_shape=[jax.ShapeDtypeStruct((n, h), jnp.float32),
                   jax.ShapeDtypeStruct((n, h), jnp.float32),
                   jax.ShapeDtypeStruct((n, cdim), jnp.float32),
                   jax.ShapeDtypeStruct((n, cdim), jnp.float32)],
    )(x, aggp, degp, W_self1, W_neigh1, b1.reshape(1, h), W_self2, W_neigh2)

    agg2_fn = _make_edge_agg(n, e, cdim, nsub=5)
    (agg2p,) = agg2_fn(z, src, dst)

    (h2,) = pl.pallas_call(
        _dense2_body,
        grid=grid,
        in_specs=[row_spec(cdim), part_spec(cdim), part_spec(LANES),
                  full_spec(1, cdim)],
        out_specs=[row_spec(cdim)],
        out_shape=[jax.ShapeDtypeStruct((n, cdim), jnp.float32)],
    )(s2, agg2p, degp, b2.reshape(1, cdim))

    return (h2, h1, h1r)
